# baseline (speedup 1.0000x reference)
import functools

import jax
import jax.numpy as jnp
from jax import lax
from jax.experimental import pallas as pl
from jax.experimental.pallas import tpu as pltpu
from jax.experimental.pallas import tpu_sc as plsc

MEM_DIM = 128
TIME_DIM = 32
IDX_CHUNK = 128


def _sc_gather(ids2, memory, lut, ts2, blk_off, n_rows_idx):
    B = n_rows_idx * IDX_CHUNK
    info = plsc.get_sparse_core_info()
    nw = info.num_cores * info.num_subcores
    b_per_w = B // nw
    chunks = b_per_w // IDX_CHUNK
    mesh = plsc.VectorSubcoreMesh(core_axis_name="c", subcore_axis_name="s")

    @functools.partial(
        pl.kernel,
        mesh=mesh,
        out_type=(
            jax.ShapeDtypeStruct((B, MEM_DIM), jnp.float32),
            jax.ShapeDtypeStruct((n_rows_idx, IDX_CHUNK), jnp.float32),
        ),
        scratch_types=[
            pltpu.VMEM((chunks, IDX_CHUNK), jnp.int32),
            pltpu.VMEM((b_per_w, MEM_DIM), jnp.float32),
            pltpu.VMEM((chunks, IDX_CHUNK), jnp.float32),
            pltpu.VMEM((chunks, IDX_CHUNK), jnp.float32),
            pltpu.VMEM((chunks, IDX_CHUNK), jnp.float32),
            pltpu.SemaphoreType.DMA,
            pltpu.SemaphoreType.DMA,
            pltpu.SemaphoreType.DMA,
        ],
    )
    def k(ids_hbm, mem_hbm, lut_hbm, ts_hbm, rows_out, d_out,
          idx_v, rows_v, lt_v, ts_v, d_v, sem_r, sem_t, sem_w):
        wid = lax.axis_index("s") * info.num_cores + lax.axis_index("c")
        for c in range(chunks):
            src = blk_off + wid + nw * c
            pltpu.sync_copy(ids_hbm.at[pl.ds(src, 1)],
                            idx_v.at[pl.ds(c, 1)])
            pltpu.sync_copy(ts_hbm.at[pl.ds(src, 1)],
                            ts_v.at[pl.ds(c, 1)])
        row_cps = []
        lut_cps = []
        for c in range(chunks):
            row_cps.append(pltpu.async_copy(
                mem_hbm.at[idx_v.at[c]],
                rows_v.at[pl.ds(c * IDX_CHUNK, IDX_CHUNK)], sem_r))
            lut_cps.append(pltpu.async_copy(
                lut_hbm.at[idx_v.at[c]], lt_v.at[c], sem_t))
        wr = []
        for c in range(chunks):
            row_cps[c].wait()
            dst = (wid + nw * c) * IDX_CHUNK
            wr.append(pltpu.async_copy(
                rows_v.at[pl.ds(c * IDX_CHUNK, IDX_CHUNK)],
                rows_out.at[pl.ds(dst, IDX_CHUNK)], sem_w))
        for cp in lut_cps:
            cp.wait()
        for c in range(chunks):
            for g in range(IDX_CHUNK // 16):
                sl = pl.ds(g * 16, 16)
                d_v[c, sl] = ts_v[c, sl] - lt_v[c, sl]
        for c in range(chunks):
            pltpu.sync_copy(d_v.at[pl.ds(c, 1)],
                            d_out.at[pl.ds(wid + nw * c, 1)])
        for cp in wr:
            cp.wait()

    return k(ids2, memory, lut, ts2)


def _tc_fuse_part(rows, delta2, W_time, b_time2, W_fusion, b_fusion2, eye,
                  B_total, blk_off, carry):
    Bh = rows.shape[0]
    BLK = 4096
    grid = Bh // BLK
    sub = BLK // IDX_CHUNK

    def body(rows_ref, d_ref, wt_ref, bt_ref, wf_ref, bf_ref, eye_ref,
             *rest):
        out_ref = rest[-1]
        i = pl.program_id(0)
        cols = [
            lax.dot_general(
                eye_ref[...], d_ref[pl.ds(sub * i + r, 1), :],
                (((1,), (1,)), ((), ())),
                preferred_element_type=jnp.float32)
            for r in range(sub)
        ]
        dcol = jnp.concatenate(cols, axis=0)
        enc = jnp.tanh(dcol * wt_ref[...] + bt_ref[...])
        h = (jnp.dot(rows_ref[...], wf_ref[:MEM_DIM, :],
                     preferred_element_type=jnp.float32)
             + jnp.dot(enc, wf_ref[MEM_DIM:, :],
                       preferred_element_type=jnp.float32)
             + bf_ref[...])
        out_ref[...] = jnp.tanh(h)

    in_specs = [
        pl.BlockSpec((BLK, MEM_DIM), lambda i: (i, 0)),
        pl.BlockSpec((Bh // IDX_CHUNK, IDX_CHUNK), lambda i: (0, 0)),
        pl.BlockSpec((1, TIME_DIM), lambda i: (0, 0)),
        pl.BlockSpec((1, TIME_DIM), lambda i: (0, 0)),
        pl.BlockSpec((MEM_DIM + TIME_DIM, MEM_DIM), lambda i: (0, 0)),
        pl.BlockSpec((1, MEM_DIM), lambda i: (0, 0)),
        pl.BlockSpec((IDX_CHUNK, IDX_CHUNK), lambda i: (0, 0)),
    ]
    args = [rows, delta2, W_time, b_time2, W_fusion, b_fusion2, eye]
    aliases = {}
    if carry is not None:
        in_specs.append(pl.BlockSpec(memory_space=pltpu.MemorySpace.HBM))
        args.append(carry)
        aliases = {7: 0}

    return pl.pallas_call(
        body,
        grid=(grid,),
        in_specs=in_specs,
        out_specs=pl.BlockSpec((BLK, MEM_DIM), lambda i: (i + blk_off, 0)),
        out_shape=jax.ShapeDtypeStruct((B_total, MEM_DIM), jnp.float32),
        input_output_aliases=aliases,
    )(*args)


def kernel(node_ids, timestamps, memory, last_update_time, W_time, b_time,
           W_fusion, b_fusion):
    B = node_ids.shape[0]
    half = B // 2
    nblk = B // IDX_CHUNK
    ids2 = node_ids.astype(jnp.int32).reshape(nblk, IDX_CHUNK)
    ts2 = timestamps.reshape(nblk, IDX_CHUNK)
    eye = jnp.eye(IDX_CHUNK, dtype=jnp.float32)
    bt2 = b_time.reshape(1, TIME_DIM)
    bf2 = b_fusion.reshape(1, MEM_DIM)
    nblk_a = (3 * nblk) // 4
    rows_a, d_a = _sc_gather(ids2, memory, last_update_time, ts2,
                             0, nblk_a)
    rows_b, d_b = _sc_gather(ids2, memory, last_update_time, ts2,
                             nblk_a, nblk - nblk_a)
    part = _tc_fuse_part(rows_a, d_a, W_time, bt2, W_fusion, bf2, eye,
                         B, 0, None)
    return _tc_fuse_part(rows_b, d_b, W_time, bt2, W_fusion, bf2, eye,
                         B, (nblk_a * IDX_CHUNK) // 4096, part)


# speedup vs baseline: 1.0946x; 1.0946x over previous
import functools

import jax
import jax.numpy as jnp
from jax import lax
from jax.experimental import pallas as pl
from jax.experimental.pallas import tpu as pltpu
from jax.experimental.pallas import tpu_sc as plsc

MEM_DIM = 128
TIME_DIM = 32
IDX_CHUNK = 128


def _sc_gather(ids2, memory, lut, ts2):
    n_rows_idx = ids2.shape[0]
    B = n_rows_idx * IDX_CHUNK
    info = plsc.get_sparse_core_info()
    nw = info.num_cores * info.num_subcores
    b_per_w = B // nw
    chunks = b_per_w // IDX_CHUNK
    mesh = plsc.VectorSubcoreMesh(core_axis_name="c", subcore_axis_name="s")

    @functools.partial(
        pl.kernel,
        mesh=mesh,
        out_type=(
            jax.ShapeDtypeStruct((B, MEM_DIM), jnp.float32),
            jax.ShapeDtypeStruct((n_rows_idx, IDX_CHUNK), jnp.float32),
        ),
        scratch_types=[
            pltpu.VMEM((chunks, IDX_CHUNK), jnp.int32),
            pltpu.VMEM((b_per_w, MEM_DIM), jnp.float32),
            pltpu.VMEM((chunks, IDX_CHUNK), jnp.float32),
            pltpu.VMEM((chunks, IDX_CHUNK), jnp.float32),
            pltpu.VMEM((chunks, IDX_CHUNK), jnp.float32),
            pltpu.SemaphoreType.DMA,
            pltpu.SemaphoreType.DMA,
            pltpu.SemaphoreType.DMA,
        ],
    )
    def k(ids_hbm, mem_hbm, lut_hbm, ts_hbm, rows_out, d_out,
          idx_v, rows_v, lt_v, ts_v, d_v, sem_r, sem_t, sem_w):
        wid = lax.axis_index("s") * info.num_cores + lax.axis_index("c")
        base = wid * b_per_w
        pltpu.sync_copy(ids_hbm.at[pl.ds(wid * chunks, chunks)], idx_v)
        pltpu.sync_copy(ts_hbm.at[pl.ds(wid * chunks, chunks)], ts_v)
        lut_cps = []
        row_cps = []
        for c in range(chunks):
            lut_cps.append(pltpu.async_copy(
                lut_hbm.at[idx_v.at[c]], lt_v.at[c], sem_t))
        for c in range(chunks):
            row_cps.append(pltpu.async_copy(
                mem_hbm.at[idx_v.at[c]],
                rows_v.at[pl.ds(c * IDX_CHUNK, IDX_CHUNK)], sem_r))
        for cp in lut_cps:
            cp.wait()
        for c in range(chunks):
            for g in range(IDX_CHUNK // 16):
                sl = pl.ds(g * 16, 16)
                d_v[c, sl] = ts_v[c, sl] - lt_v[c, sl]
        pltpu.sync_copy(d_v, d_out.at[pl.ds(wid * chunks, chunks)])
        wr = []
        for c in range(chunks):
            row_cps[c].wait()
            wr.append(pltpu.async_copy(
                rows_v.at[pl.ds(c * IDX_CHUNK, IDX_CHUNK)],
                rows_out.at[pl.ds(base + c * IDX_CHUNK, IDX_CHUNK)], sem_w))
        for cp in wr:
            cp.wait()

    return k(ids2, memory, lut, ts2)


def _tc_fuse(rows, delta2, W_time, b_time2, W_fusion, b_fusion2, eye):
    B = rows.shape[0]
    BLK = 8192
    grid = B // BLK
    sub = BLK // IDX_CHUNK

    def body(rows_ref, d_ref, wt_ref, bt_ref, wf_ref, bf_ref, eye_ref,
             out_ref):
        i = pl.program_id(0)
        cols = [
            lax.dot_general(
                eye_ref[...], d_ref[pl.ds(sub * i + r, 1), :],
                (((1,), (1,)), ((), ())),
                preferred_element_type=jnp.float32)
            for r in range(sub)
        ]
        dcol = jnp.concatenate(cols, axis=0)
        enc = jnp.tanh(dcol * wt_ref[...] + bt_ref[...])
        h = (jnp.dot(rows_ref[...], wf_ref[:MEM_DIM, :],
                     preferred_element_type=jnp.float32)
             + jnp.dot(enc, wf_ref[MEM_DIM:, :],
                       preferred_element_type=jnp.float32)
             + bf_ref[...])
        out_ref[...] = jnp.tanh(h)

    return pl.pallas_call(
        body,
        grid=(grid,),
        in_specs=[
            pl.BlockSpec((BLK, MEM_DIM), lambda i: (i, 0)),
            pl.BlockSpec((B // IDX_CHUNK, IDX_CHUNK), lambda i: (0, 0)),
            pl.BlockSpec((1, TIME_DIM), lambda i: (0, 0)),
            pl.BlockSpec((1, TIME_DIM), lambda i: (0, 0)),
            pl.BlockSpec((MEM_DIM + TIME_DIM, MEM_DIM), lambda i: (0, 0)),
            pl.BlockSpec((1, MEM_DIM), lambda i: (0, 0)),
            pl.BlockSpec((IDX_CHUNK, IDX_CHUNK), lambda i: (0, 0)),
        ],
        out_specs=pl.BlockSpec((BLK, MEM_DIM), lambda i: (i, 0)),
        out_shape=jax.ShapeDtypeStruct((B, MEM_DIM), jnp.float32),
    )(rows, delta2, W_time, b_time2, W_fusion, b_fusion2, eye)


def kernel(node_ids, timestamps, memory, last_update_time, W_time, b_time,
           W_fusion, b_fusion):
    B = node_ids.shape[0]
    ids2 = node_ids.astype(jnp.int32).reshape(B // IDX_CHUNK, IDX_CHUNK)
    ts2 = timestamps.reshape(B // IDX_CHUNK, IDX_CHUNK)
    rows, delta2 = _sc_gather(ids2, memory, last_update_time, ts2)
    eye = jnp.eye(IDX_CHUNK, dtype=jnp.float32)
    return _tc_fuse(rows, delta2, W_time, b_time.reshape(1, TIME_DIM),
                    W_fusion, b_fusion.reshape(1, MEM_DIM), eye)


# speedup vs baseline: 1.1078x; 1.0121x over previous
import functools

import jax
import jax.numpy as jnp
from jax import lax
from jax.experimental import pallas as pl
from jax.experimental.pallas import tpu as pltpu
from jax.experimental.pallas import tpu_sc as plsc

MEM_DIM = 128
TIME_DIM = 32
IDX_CHUNK = 128


def _sc_gather(ids2, memory, lut, ts2):
    n_rows_idx = ids2.shape[0]
    B = n_rows_idx * IDX_CHUNK
    info = plsc.get_sparse_core_info()
    nw = info.num_cores * info.num_subcores
    b_per_w = B // nw
    chunks = b_per_w // IDX_CHUNK
    mesh = plsc.VectorSubcoreMesh(core_axis_name="c", subcore_axis_name="s")

    @functools.partial(
        pl.kernel,
        mesh=mesh,
        out_type=(
            jax.ShapeDtypeStruct((B, MEM_DIM), jnp.float32),
            jax.ShapeDtypeStruct((n_rows_idx, IDX_CHUNK), jnp.float32),
        ),
        scratch_types=[
            pltpu.VMEM((chunks, IDX_CHUNK), jnp.int32),
            pltpu.VMEM((b_per_w, MEM_DIM), jnp.float32),
            pltpu.VMEM((chunks, IDX_CHUNK), jnp.float32),
            pltpu.VMEM((chunks, IDX_CHUNK), jnp.float32),
            pltpu.VMEM((chunks, IDX_CHUNK), jnp.float32),
            pltpu.SemaphoreType.DMA,
            pltpu.SemaphoreType.DMA,
            pltpu.SemaphoreType.DMA,
        ],
    )
    def k(ids_hbm, mem_hbm, lut_hbm, ts_hbm, rows_out, d_out,
          idx_v, rows_v, lt_v, ts_v, d_v, sem_r, sem_t, sem_w):
        wid = lax.axis_index("s") * info.num_cores + lax.axis_index("c")
        base = wid * b_per_w
        ts_cp = pltpu.async_copy(
            ts_hbm.at[pl.ds(wid * chunks, chunks)], ts_v, sem_w)
        pltpu.sync_copy(ids_hbm.at[pl.ds(wid * chunks, chunks)], idx_v)
        lut_cps = []
        row_cps = []
        for c in range(chunks):
            lut_cps.append(pltpu.async_copy(
                lut_hbm.at[idx_v.at[c]], lt_v.at[c], sem_t))
        for c in range(chunks):
            row_cps.append(pltpu.async_copy(
                mem_hbm.at[idx_v.at[c]],
                rows_v.at[pl.ds(c * IDX_CHUNK, IDX_CHUNK)], sem_r))
        ts_cp.wait()
        for cp in lut_cps:
            cp.wait()
        for c in range(chunks):
            for g in range(IDX_CHUNK // 16):
                sl = pl.ds(g * 16, 16)
                d_v[c, sl] = ts_v[c, sl] - lt_v[c, sl]
        pltpu.sync_copy(d_v, d_out.at[pl.ds(wid * chunks, chunks)])
        wr = []
        for c in range(chunks):
            row_cps[c].wait()
            wr.append(pltpu.async_copy(
                rows_v.at[pl.ds(c * IDX_CHUNK, IDX_CHUNK)],
                rows_out.at[pl.ds(base + c * IDX_CHUNK, IDX_CHUNK)], sem_w))
        for cp in wr:
            cp.wait()

    return k(ids2, memory, lut, ts2)


def _tc_fuse(rows, delta2, W_time, b_time2, W_fusion, b_fusion2, eye):
    B = rows.shape[0]
    BLK = 8192
    grid = B // BLK
    sub = BLK // IDX_CHUNK

    def body(rows_ref, d_ref, wt_ref, bt_ref, wf_ref, bf_ref, eye_ref,
             out_ref):
        i = pl.program_id(0)
        cols = [
            lax.dot_general(
                eye_ref[...], d_ref[pl.ds(sub * i + r, 1), :],
                (((1,), (1,)), ((), ())),
                preferred_element_type=jnp.float32)
            for r in range(sub)
        ]
        dcol = jnp.concatenate(cols, axis=0)
        enc = jnp.tanh(dcol * wt_ref[...] + bt_ref[...])
        h = (jnp.dot(rows_ref[...], wf_ref[:MEM_DIM, :],
                     preferred_element_type=jnp.float32)
             + jnp.dot(enc, wf_ref[MEM_DIM:, :],
                       preferred_element_type=jnp.float32)
             + bf_ref[...])
        out_ref[...] = jnp.tanh(h)

    return pl.pallas_call(
        body,
        grid=(grid,),
        in_specs=[
            pl.BlockSpec((BLK, MEM_DIM), lambda i: (i, 0)),
            pl.BlockSpec((B // IDX_CHUNK, IDX_CHUNK), lambda i: (0, 0)),
            pl.BlockSpec((1, TIME_DIM), lambda i: (0, 0)),
            pl.BlockSpec((1, TIME_DIM), lambda i: (0, 0)),
            pl.BlockSpec((MEM_DIM + TIME_DIM, MEM_DIM), lambda i: (0, 0)),
            pl.BlockSpec((1, MEM_DIM), lambda i: (0, 0)),
            pl.BlockSpec((IDX_CHUNK, IDX_CHUNK), lambda i: (0, 0)),
        ],
        out_specs=pl.BlockSpec((BLK, MEM_DIM), lambda i: (i, 0)),
        out_shape=jax.ShapeDtypeStruct((B, MEM_DIM), jnp.float32),
    )(rows, delta2, W_time, b_time2, W_fusion, b_fusion2, eye)


def kernel(node_ids, timestamps, memory, last_update_time, W_time, b_time,
           W_fusion, b_fusion):
    B = node_ids.shape[0]
    ids2 = node_ids.astype(jnp.int32).reshape(B // IDX_CHUNK, IDX_CHUNK)
    ts2 = timestamps.reshape(B // IDX_CHUNK, IDX_CHUNK)
    rows, delta2 = _sc_gather(ids2, memory, last_update_time, ts2)
    eye = jnp.eye(IDX_CHUNK, dtype=jnp.float32)
    return _tc_fuse(rows, delta2, W_time, b_time.reshape(1, TIME_DIM),
                    W_fusion, b_fusion.reshape(1, MEM_DIM), eye)


# speedup vs baseline: 1.1090x; 1.0011x over previous
import functools

import jax
import jax.numpy as jnp
from jax import lax
from jax.experimental import pallas as pl
from jax.experimental.pallas import tpu as pltpu
from jax.experimental.pallas import tpu_sc as plsc

MEM_DIM = 128
TIME_DIM = 32
IDX_CHUNK = 128


def _sc_gather(ids2, memory, lut, ts2):
    n_rows_idx = ids2.shape[0]
    B = n_rows_idx * IDX_CHUNK
    info = plsc.get_sparse_core_info()
    nw = info.num_cores * info.num_subcores
    b_per_w = B // nw
    chunks = b_per_w // IDX_CHUNK
    mesh = plsc.VectorSubcoreMesh(core_axis_name="c", subcore_axis_name="s")

    @functools.partial(
        pl.kernel,
        mesh=mesh,
        out_type=(
            jax.ShapeDtypeStruct((B, MEM_DIM), jnp.float32),
            jax.ShapeDtypeStruct((n_rows_idx, IDX_CHUNK), jnp.float32),
        ),
        scratch_types=[
            pltpu.VMEM((chunks, IDX_CHUNK), jnp.int32),
            pltpu.VMEM((b_per_w, MEM_DIM), jnp.float32),
            pltpu.VMEM((chunks, IDX_CHUNK), jnp.float32),
            pltpu.VMEM((chunks, IDX_CHUNK), jnp.float32),
            pltpu.VMEM((chunks, IDX_CHUNK), jnp.float32),
            pltpu.SemaphoreType.DMA,
            pltpu.SemaphoreType.DMA,
            pltpu.SemaphoreType.DMA,
        ],
    )
    def k(ids_hbm, mem_hbm, lut_hbm, ts_hbm, rows_out, d_out,
          idx_v, rows_v, lt_v, ts_v, d_v, sem_r, sem_t, sem_w):
        wid = lax.axis_index("s") * info.num_cores + lax.axis_index("c")
        base = wid * b_per_w
        ts_cp = pltpu.async_copy(
            ts_hbm.at[pl.ds(wid * chunks, chunks)], ts_v, sem_w)
        pltpu.sync_copy(ids_hbm.at[pl.ds(wid * chunks, chunks)], idx_v)
        lut_cps = []
        row_cps = []
        for c in range(chunks):
            row_cps.append(pltpu.async_copy(
                mem_hbm.at[idx_v.at[c]],
                rows_v.at[pl.ds(c * IDX_CHUNK, IDX_CHUNK)], sem_r))
        for c in range(chunks):
            lut_cps.append(pltpu.async_copy(
                lut_hbm.at[idx_v.at[c]], lt_v.at[c], sem_t))
        ts_cp.wait()
        for cp in lut_cps:
            cp.wait()
        for c in range(chunks):
            for g in range(IDX_CHUNK // 16):
                sl = pl.ds(g * 16, 16)
                d_v[c, sl] = ts_v[c, sl] - lt_v[c, sl]
        d_cp = pltpu.async_copy(
            d_v, d_out.at[pl.ds(wid * chunks, chunks)], sem_t)
        wr = []
        for c in range(chunks):
            row_cps[c].wait()
            wr.append(pltpu.async_copy(
                rows_v.at[pl.ds(c * IDX_CHUNK, IDX_CHUNK)],
                rows_out.at[pl.ds(base + c * IDX_CHUNK, IDX_CHUNK)], sem_w))
        d_cp.wait()
        for cp in wr:
            cp.wait()

    return k(ids2, memory, lut, ts2)


def _tc_fuse(rows, delta2, W_time, b_time2, W_fusion, b_fusion2, eye):
    B = rows.shape[0]
    BLK = 8192
    grid = B // BLK
    sub = BLK // IDX_CHUNK

    def body(rows_ref, d_ref, wt_ref, bt_ref, wf_ref, bf_ref, eye_ref,
             out_ref):
        i = pl.program_id(0)
        cols = [
            lax.dot_general(
                eye_ref[...], d_ref[pl.ds(sub * i + r, 1), :],
                (((1,), (1,)), ((), ())),
                preferred_element_type=jnp.float32)
            for r in range(sub)
        ]
        dcol = jnp.concatenate(cols, axis=0)
        enc = jnp.tanh(dcol * wt_ref[...] + bt_ref[...])
        h = (jnp.dot(rows_ref[...], wf_ref[:MEM_DIM, :],
                     preferred_element_type=jnp.float32)
             + jnp.dot(enc, wf_ref[MEM_DIM:, :],
                       preferred_element_type=jnp.float32)
             + bf_ref[...])
        out_ref[...] = jnp.tanh(h)

    return pl.pallas_call(
        body,
        grid=(grid,),
        in_specs=[
            pl.BlockSpec((BLK, MEM_DIM), lambda i: (i, 0)),
            pl.BlockSpec((B // IDX_CHUNK, IDX_CHUNK), lambda i: (0, 0)),
            pl.BlockSpec((1, TIME_DIM), lambda i: (0, 0)),
            pl.BlockSpec((1, TIME_DIM), lambda i: (0, 0)),
            pl.BlockSpec((MEM_DIM + TIME_DIM, MEM_DIM), lambda i: (0, 0)),
            pl.BlockSpec((1, MEM_DIM), lambda i: (0, 0)),
            pl.BlockSpec((IDX_CHUNK, IDX_CHUNK), lambda i: (0, 0)),
        ],
        out_specs=pl.BlockSpec((BLK, MEM_DIM), lambda i: (i, 0)),
        out_shape=jax.ShapeDtypeStruct((B, MEM_DIM), jnp.float32),
    )(rows, delta2, W_time, b_time2, W_fusion, b_fusion2, eye)


def kernel(node_ids, timestamps, memory, last_update_time, W_time, b_time,
           W_fusion, b_fusion):
    B = node_ids.shape[0]
    ids2 = node_ids.astype(jnp.int32).reshape(B // IDX_CHUNK, IDX_CHUNK)
    ts2 = timestamps.reshape(B // IDX_CHUNK, IDX_CHUNK)
    rows, delta2 = _sc_gather(ids2, memory, last_update_time, ts2)
    eye = jnp.eye(IDX_CHUNK, dtype=jnp.float32)
    return _tc_fuse(rows, delta2, W_time, b_time.reshape(1, TIME_DIM),
                    W_fusion, b_fusion.reshape(1, MEM_DIM), eye)
